# baseline (speedup 1.0000x reference)
import jax
import jax.numpy as jnp
from jax import lax
from jax.experimental import pallas as pl
from jax.experimental.pallas import tpu as pltpu
from jax.experimental.pallas import tpu_sc as plsc

_NC = 2
_NS = 16
_L = 16
_NW = _NC * _NS

_N = 16384
_C = 16
_RPW = _N // _NW
_RPC = _N // _NC
_HALF = _RPW // 2
_BULK = _RPC // 2
_BLOCKS = _RPW // _L
_BIG = 1 << 20


def _body(x_hbm, keys_hbm, values_hbm, out_hbm, kv_v, vv_v, t_v, x_v, o_v,
          x_spm, sem_k, sem_v, sem_x):
    cid = lax.axis_index("c")
    sid = lax.axis_index("s")
    core0 = cid * _RPC
    front = core0 + sid * _HALF
    back = core0 + _BULK + sid * _HALF

    x_cp = pltpu.async_copy(
        x_hbm.at[pl.ds(back * _C, _HALF * _C)],
        x_v.at[pl.ds(_HALF * _C, _HALF * _C)], sem_x)
    k_cp = pltpu.async_copy(keys_hbm, kv_v, sem_k)
    v_cp = pltpu.async_copy(values_hbm, vv_v, sem_v)

    @pl.when(sid == 0)
    def _():
        pltpu.sync_copy(x_hbm.at[pl.ds(core0 * _C, _BULK * _C)], x_spm)

    k_cp.wait()
    v_cp.wait()

    lane = lax.iota(jnp.int32, _L)

    kvec = kv_v[...]
    pos = jnp.where(kvec[0] < lane, 1, 0).astype(jnp.int32)
    for k in range(1, _C):
        pos = pos + jnp.where(kvec[k] < lane, 1, 0).astype(jnp.int32)
    pos_c = jnp.minimum(pos, _C - 1)
    key_at = plsc.load_gather(kv_v, [pos_c])
    val_at = plsc.load_gather(vv_v, [pos_c])
    t_v[...] = jnp.where(key_at == lane, val_at, jnp.float32(-1.0))

    plsc.subcore_barrier()
    pltpu.sync_copy(x_spm.at[pl.ds(sid * _HALF * _C, _HALF * _C)],
                    x_v.at[pl.ds(0, _HALF * _C)])
    x_cp.wait()

    cols = [jnp.bitwise_and(lane + j, _C - 1) for j in range(_C)]
    row0 = lane * _C

    @plsc.parallel_loop(0, _BLOCKS, unroll=2)
    def _blk(b):
        addr0 = b * (_L * _C) + row0
        vs = [plsc.load_gather(x_v, [addr0 + cols[j]]) for j in range(_C)]
        m = vs
        while len(m) > 1:
            m = [jnp.maximum(m[i], m[i + 1]) for i in range(0, len(m), 2)]
        mx = m[0]
        bi = jnp.where(vs[0] == mx, cols[0], _BIG)
        for j in range(1, _C):
            bi = jnp.minimum(bi, jnp.where(vs[j] == mx, cols[j], _BIG))
        res = plsc.load_gather(t_v, [bi])
        o_v[pl.ds(b * _L, _L)] = res

    pltpu.sync_copy(o_v.at[pl.ds(0, _HALF)], out_hbm.at[pl.ds(front, _HALF)])
    pltpu.sync_copy(o_v.at[pl.ds(_HALF, _HALF)], out_hbm.at[pl.ds(back, _HALF)])


@jax.jit
def _run(x_flat, keys_i32, values):
    return pl.kernel(
        _body,
        out_type=jax.ShapeDtypeStruct((_N,), jnp.float32),
        mesh=plsc.VectorSubcoreMesh(core_axis_name="c", subcore_axis_name="s"),
        compiler_params=pltpu.CompilerParams(needs_layout_passes=False),
        scratch_types=[
            pltpu.VMEM((_C,), jnp.int32),
            pltpu.VMEM((_C,), jnp.float32),
            pltpu.VMEM((_C,), jnp.float32),
            pltpu.VMEM((_RPW * _C,), jnp.float32),
            pltpu.VMEM((_RPW,), jnp.float32),
            pltpu.VMEM_SHARED((_BULK * _C,), jnp.float32),
            pltpu.SemaphoreType.DMA,
            pltpu.SemaphoreType.DMA,
            pltpu.SemaphoreType.DMA,
        ],
    )(x_flat, keys_i32, values)


def kernel(tensor_input, keys, values):
    x_flat = jnp.reshape(tensor_input, (-1,))
    return _run(x_flat, keys.astype(jnp.int32), values)


# speedup vs baseline: 1.0050x; 1.0050x over previous
import jax
import jax.numpy as jnp
from jax import lax
from jax.experimental import pallas as pl
from jax.experimental.pallas import tpu as pltpu
from jax.experimental.pallas import tpu_sc as plsc

_NC = 2
_NS = 16
_L = 16
_NW = _NC * _NS

_N = 16384
_C = 16
_RPW = _N // _NW
_BLOCKS = _RPW // _L
_BIG = 1 << 20


def _body(x_hbm, keys_hbm, values_hbm, out_hbm, kv_v, vv_v, t_v, x_v, o_v,
          sem_k, sem_v, sem_x, sem_x2, sem_o):
    cid = lax.axis_index("c")
    sid = lax.axis_index("s")
    wid = sid * _NC + cid
    base = wid * _RPW
    halfw = _RPW * _C // 2

    x_cp = pltpu.async_copy(x_hbm.at[pl.ds(base * _C, halfw)],
                            x_v.at[pl.ds(0, halfw)], sem_x)
    x2_cp = pltpu.async_copy(x_hbm.at[pl.ds(base * _C + halfw, halfw)],
                             x_v.at[pl.ds(halfw, halfw)], sem_x2)
    k_cp = pltpu.async_copy(keys_hbm, kv_v, sem_k)
    v_cp = pltpu.async_copy(values_hbm, vv_v, sem_v)
    k_cp.wait()
    v_cp.wait()

    lane = lax.iota(jnp.int32, _L)

    kvec = kv_v[...]
    pos = jnp.where(kvec[0] < lane, 1, 0).astype(jnp.int32)
    for k in range(1, _C):
        pos = pos + jnp.where(kvec[k] < lane, 1, 0).astype(jnp.int32)
    pos_c = jnp.minimum(pos, _C - 1)
    key_at = plsc.load_gather(kv_v, [pos_c])
    val_at = plsc.load_gather(vv_v, [pos_c])
    t_v[...] = jnp.where(key_at == lane, val_at, jnp.float32(-1.0))

    cols = [jnp.bitwise_and(lane + j, _C - 1) for j in range(_C)]
    row0 = lane * _C

    def _blk(b):
        addr0 = b * (_L * _C) + row0
        vs = [plsc.load_gather(x_v, [addr0 + cols[j]]) for j in range(_C)]
        m = vs
        while len(m) > 1:
            m = [jnp.maximum(m[i], m[i + 1]) for i in range(0, len(m), 2)]
        mx = m[0]
        bi = jnp.where(vs[0] == mx, cols[0], _BIG)
        for j in range(1, _C):
            bi = jnp.minimum(bi, jnp.where(vs[j] == mx, cols[j], _BIG))
        res = plsc.load_gather(t_v, [bi])
        o_v[pl.ds(b * _L, _L)] = res

    x_cp.wait()
    plsc.parallel_loop(0, _BLOCKS // 2, unroll=2)(_blk)
    o_cp = pltpu.async_copy(o_v.at[pl.ds(0, _RPW // 2)],
                            out_hbm.at[pl.ds(base, _RPW // 2)], sem_o)
    x2_cp.wait()
    plsc.parallel_loop(_BLOCKS // 2, _BLOCKS, unroll=2)(_blk)
    o_cp.wait()
    pltpu.sync_copy(o_v.at[pl.ds(_RPW // 2, _RPW // 2)],
                    out_hbm.at[pl.ds(base + _RPW // 2, _RPW // 2)])


@jax.jit
def _run(x_flat, keys_i32, values):
    return pl.kernel(
        _body,
        out_type=jax.ShapeDtypeStruct((_N,), jnp.float32),
        mesh=plsc.VectorSubcoreMesh(core_axis_name="c", subcore_axis_name="s"),
        compiler_params=pltpu.CompilerParams(needs_layout_passes=False),
        scratch_types=[
            pltpu.VMEM((_C,), jnp.int32),
            pltpu.VMEM((_C,), jnp.float32),
            pltpu.VMEM((_C,), jnp.float32),
            pltpu.VMEM((_RPW * _C,), jnp.float32),
            pltpu.VMEM((_RPW,), jnp.float32),
            pltpu.SemaphoreType.DMA,
            pltpu.SemaphoreType.DMA,
            pltpu.SemaphoreType.DMA,
            pltpu.SemaphoreType.DMA,
            pltpu.SemaphoreType.DMA,
        ],
    )(x_flat, keys_i32, values)


def kernel(tensor_input, keys, values):
    x_flat = jnp.reshape(tensor_input, (-1,))
    return _run(x_flat, keys.astype(jnp.int32), values)
